# baseline (device time: 12769 ns/iter reference)
import jax
import jax.numpy as jnp
from jax import lax
from jax.experimental import pallas as pl
from jax.experimental.pallas import tpu as pltpu

N_DEV = 4
CHUNKS = (128, 128)
HOP_ORDER = (2, 1, 3)


def kernel(x, Wg, Wu, Wd):
    m, k = x.shape
    hdim = Wg.shape[1]
    d = Wd.shape[1]
    n_chunk = len(CHUNKS)
    offs = [sum(CHUNKS[:c]) for c in range(n_chunk)]

    def body(x_hbm, wg_hbm, wu_hbm, wd_hbm, out_ref, *scratch):
        xv, wgv, wuv, wdv = scratch[:4]
        in_sems = scratch[4]
        send_refs = scratch[5:5 + n_chunk]
        comm_refs = scratch[5 + n_chunk:5 + 2 * n_chunk]
        send_sems, recv_sems = scratch[5 + 2 * n_chunk:]

        my_pos = lax.axis_index("i")

        barrier_sem = pltpu.get_barrier_semaphore()
        for h in range(1, N_DEV):
            pl.semaphore_signal(
                barrier_sem, inc=1,
                device_id=(lax.rem(my_pos + h, N_DEV),),
                device_id_type=pl.DeviceIdType.MESH,
            )

        cps = [
            pltpu.make_async_copy(src, dst, in_sems.at[i])
            for i, (src, dst) in enumerate(
                [(x_hbm, xv), (wg_hbm, wgv), (wu_hbm, wuv), (wd_hbm, wdv)]
            )
        ]
        for cp in cps:
            cp.start()

        cps[0].wait()
        xb = xv[...].astype(jnp.bfloat16)
        cps[1].wait()
        gate = jnp.dot(xb, wgv[...].astype(jnp.bfloat16),
                       preferred_element_type=jnp.float32)
        cps[2].wait()
        up = jnp.dot(xb, wuv[...].astype(jnp.bfloat16),
                     preferred_element_type=jnp.float32)
        hidden = (gate * (up * jax.nn.sigmoid(up))).astype(jnp.bfloat16)
        cps[3].wait()
        wdb = wdv[...].astype(jnp.bfloat16)

        partials = []
        rdmas = []
        for c, sz in enumerate(CHUNKS):
            p = jnp.dot(hidden[offs[c]:offs[c] + sz, :], wdb,
                        preferred_element_type=jnp.float32)
            partials.append(p)
            send_refs[c][...] = p.astype(jnp.bfloat16)
            if c == 0:
                pl.semaphore_wait(barrier_sem, N_DEV - 1)
            chunk_rdmas = {}
            for h in HOP_ORDER:
                rdma = pltpu.make_async_remote_copy(
                    src_ref=send_refs[c],
                    dst_ref=comm_refs[c].at[h - 1],
                    send_sem=send_sems.at[h - 1, c],
                    recv_sem=recv_sems.at[h - 1, c],
                    device_id=(lax.rem(my_pos + h, N_DEV),),
                    device_id_type=pl.DeviceIdType.MESH,
                )
                rdma.start()
                chunk_rdmas[h] = rdma
            rdmas.append(chunk_rdmas)

        for c, sz in enumerate(CHUNKS):
            rdmas[c][1].wait_recv()
            rdmas[c][3].wait_recv()
            near = (partials[c]
                    + comm_refs[c][0].astype(jnp.float32)
                    + comm_refs[c][2].astype(jnp.float32))
            rdmas[c][2].wait_recv()
            out_ref[offs[c]:offs[c] + sz, :] = (
                near + comm_refs[c][1].astype(jnp.float32)
            ).astype(jnp.bfloat16)

        for chunk_rdmas in rdmas:
            for rdma in chunk_rdmas.values():
                rdma.wait_send()

    return pl.pallas_call(
        body,
        out_shape=jax.ShapeDtypeStruct((m, d), jnp.bfloat16),
        in_specs=[pl.BlockSpec(memory_space=pl.ANY)] * 4,
        out_specs=pl.BlockSpec(memory_space=pltpu.VMEM),
        scratch_shapes=(
            [
                pltpu.VMEM((m, k), jnp.float32),
                pltpu.VMEM((k, hdim), jnp.float32),
                pltpu.VMEM((k, hdim), jnp.float32),
                pltpu.VMEM((hdim, d), jnp.float32),
                pltpu.SemaphoreType.DMA((4,)),
            ]
            + [pltpu.VMEM((sz, d), jnp.bfloat16) for sz in CHUNKS]
            + [pltpu.VMEM((N_DEV - 1, sz, d), jnp.bfloat16) for sz in CHUNKS]
            + [
                pltpu.SemaphoreType.DMA((N_DEV - 1, n_chunk)),
                pltpu.SemaphoreType.DMA((N_DEV - 1, n_chunk)),
            ]
        ),
        compiler_params=pltpu.CompilerParams(collective_id=0),
    )(x, Wg, Wu, Wd)
